# Initial kernel scaffold; baseline (speedup 1.0000x reference)
import functools

import jax
import jax.numpy as jnp
from jax.experimental import pallas as pl
from jax.experimental.pallas import tpu as pltpu

_K = 8
_ROWS = 512


def _body(q_ref, yt_ref, out_ref, *, n):
    bi = pl.program_id(0)
    ji = pl.program_id(1)

    q = q_ref[0]
    yt = yt_ref[0]

    sq_q = jnp.sum(q * q, axis=1, keepdims=True)
    sq_y = jnp.sum(yt * yt, axis=0, keepdims=True)
    cross = jax.lax.dot_general(
        q, yt, (((1,), (0,)), ((), ())),
        preferred_element_type=jnp.float32)
    d2 = jnp.maximum(sq_q + sq_y - 2.0 * cross, 0.0)

    inf = jnp.float32(jnp.inf)
    acc = jnp.zeros((q.shape[0], 1), jnp.float32)
    needed = jnp.full((q.shape[0], 1), float(_K), jnp.float32)
    for _ in range(_K):
        m = jnp.min(d2, axis=1, keepdims=True)
        eq = d2 == m
        cnt = jnp.sum(eq.astype(jnp.float32), axis=1, keepdims=True)
        take = jnp.minimum(cnt, needed)
        root = jnp.where(m > 0, jnp.sqrt(jnp.where(m > 0, m, 1.0)), 0.0)
        acc = acc + jnp.where(take > 0, take * root, 0.0)
        needed = needed - take
        d2 = jnp.where(eq, inf, d2)

    @pl.when(jnp.logical_and(bi == 0, ji == 0))
    def _():
        out_ref[0, 0] = 0.0

    out_ref[0, 0] += jnp.sum(acc)


def kernel(generated):
    generated = generated.astype(jnp.float32)
    b, c, h, w = generated.shape
    n = h * w
    yt = generated.reshape(b, c, n)
    q = jnp.transpose(yt, (0, 2, 1))

    nb = n // _ROWS
    total = pl.pallas_call(
        functools.partial(_body, n=n),
        grid=(b, nb),
        in_specs=[
            pl.BlockSpec((1, _ROWS, c), lambda i, j: (i, j, 0)),
            pl.BlockSpec((1, c, n), lambda i, j: (i, 0, 0)),
        ],
        out_specs=pl.BlockSpec((1, 1), lambda i, j: (0, 0)),
        out_shape=jax.ShapeDtypeStruct((1, 1), jnp.float32),
    )(q, yt)

    return -total[0, 0] / jnp.float32(b * n * _K)


# speedup vs baseline: 32.3505x; 32.3505x over previous
import functools

import jax
import jax.numpy as jnp
from jax.experimental import pallas as pl
from jax.experimental.pallas import tpu as pltpu

_K = 8
_ROWS = 512


def _body(q_ref, yt_ref, out_ref, *, n):
    bi = pl.program_id(0)
    ji = pl.program_id(1)

    q = q_ref[0]
    yt = yt_ref[0]

    sq_q = jnp.sum(q * q, axis=1, keepdims=True)
    sq_y = jnp.sum(yt * yt, axis=0, keepdims=True)
    cross = jax.lax.dot_general(
        q, yt, (((1,), (0,)), ((), ())),
        preferred_element_type=jnp.float32)
    d2 = jnp.maximum(sq_q + sq_y - 2.0 * cross, 0.0)

    inf = jnp.float32(jnp.inf)
    acc = jnp.zeros((q.shape[0], 1), jnp.float32)
    needed = jnp.full((q.shape[0], 1), float(_K), jnp.float32)
    for _ in range(_K):
        m = jnp.min(d2, axis=1, keepdims=True)
        eq = d2 == m
        cnt = jnp.sum(eq.astype(jnp.float32), axis=1, keepdims=True)
        take = jnp.minimum(cnt, needed)
        root = jnp.where(m > 0, jnp.sqrt(jnp.where(m > 0, m, 1.0)), 0.0)
        acc = acc + jnp.where(take > 0, take * root, 0.0)
        needed = needed - take
        d2 = jnp.where(eq, inf, d2)

    @pl.when(jnp.logical_and(bi == 0, ji == 0))
    def _():
        out_ref[...] = jnp.zeros_like(out_ref)

    out_ref[...] += acc


def kernel(generated):
    generated = generated.astype(jnp.float32)
    b, c, h, w = generated.shape
    n = h * w
    yt = generated.reshape(b, c, n)
    q = jnp.transpose(yt, (0, 2, 1))

    nb = n // _ROWS
    total = pl.pallas_call(
        functools.partial(_body, n=n),
        grid=(b, nb),
        in_specs=[
            pl.BlockSpec((1, _ROWS, c), lambda i, j: (i, j, 0)),
            pl.BlockSpec((1, c, n), lambda i, j: (i, 0, 0)),
        ],
        out_specs=pl.BlockSpec((_ROWS, 1), lambda i, j: (0, 0)),
        out_shape=jax.ShapeDtypeStruct((_ROWS, 1), jnp.float32),
    )(q, yt)

    return -jnp.sum(total) / jnp.float32(b * n * _K)
